# baseline (speedup 1.0000x reference)
import jax
import jax.numpy as jnp
from jax.experimental import pallas as pl

_N = 10000
_D = 128
_LAYERS = 3
_NB = 1000
_EB = 2000


def _prologue_body(h_ref, w1a_ref, w1b_ref, b1_ref, r1w1_ref, r1b1_ref,
                   r1w2_ref, r1b2_ref, a_ref, b_ref, g_ref):
    h = h_ref[...]
    a_ref[...] = jnp.dot(h, w1a_ref[...], preferred_element_type=jnp.float32) + b1_ref[...]
    b_ref[...] = jnp.dot(h, w1b_ref[...], preferred_element_type=jnp.float32)
    t = jax.nn.relu(jnp.dot(h, r1w1_ref[...], preferred_element_type=jnp.float32) + r1b1_ref[...])
    g_ref[...] = jnp.dot(t, r1w2_ref[...], preferred_element_type=jnp.float32) + r1b2_ref[...]


def _edge_body(z_ref, w2_ref, b2_ref, f_ref):
    f_ref[...] = (jnp.dot(jax.nn.relu(z_ref[...]), w2_ref[...],
                          preferred_element_type=jnp.float32) + b2_ref[...])


def _update_body(m_ref, g_ref, mask_ref, h_ref, wua_ref, wub_ref, ub1_ref,
                 uw2_ref, ub2_ref, lng_ref, lnb_ref, o_ref):
    m = m_ref[...]
    g = g_ref[...]
    mask = mask_ref[...]
    m = jnp.maximum(m, jnp.where(mask > 0.0, g, -jnp.inf))
    h = h_ref[...]
    x = h + m
    t = jax.nn.relu(jnp.dot(x, wua_ref[...], preferred_element_type=jnp.float32)
                    + jnp.dot(h, wub_ref[...], preferred_element_type=jnp.float32)
                    + ub1_ref[...])
    u = jnp.dot(t, uw2_ref[...], preferred_element_type=jnp.float32) + ub2_ref[...]
    mu = jnp.mean(u, axis=-1, keepdims=True)
    var = jnp.mean((u - mu) ** 2, axis=-1, keepdims=True)
    u = (u - mu) * jax.lax.rsqrt(var + 1e-5) * lng_ref[...] + lnb_ref[...]
    o_ref[...] = h + u


def _full(block):
    return pl.BlockSpec(block, lambda i: (0, 0))


def _rows(block):
    return pl.BlockSpec(block, lambda i: (i, 0))


def _prologue(h, w1a, w1b, b1, r1w1, r1b1, r1w2, r1b2):
    return pl.pallas_call(
        _prologue_body,
        grid=(_N // _NB,),
        in_specs=[_rows((_NB, _D)), _full((_D, 2 * _D)), _full((_D, 2 * _D)),
                  _full((1, 2 * _D)), _full((_D, _D)), _full((1, _D)),
                  _full((_D, _D)), _full((1, _D))],
        out_specs=[_rows((_NB, 2 * _D)), _rows((_NB, 2 * _D)), _rows((_NB, _D))],
        out_shape=[jax.ShapeDtypeStruct((_N, 2 * _D), jnp.float32),
                   jax.ShapeDtypeStruct((_N, 2 * _D), jnp.float32),
                   jax.ShapeDtypeStruct((_N, _D), jnp.float32)],
    )(h, w1a, w1b, b1, r1w1, r1b1, r1w2, r1b2)


def _edge_mlp(z, w2, b2):
    e = z.shape[0]
    return pl.pallas_call(
        _edge_body,
        grid=(e // _EB,),
        in_specs=[_rows((_EB, 2 * _D)), _full((2 * _D, 2 * _D)), _full((1, 2 * _D))],
        out_specs=_rows((_EB, 2 * _D)),
        out_shape=jax.ShapeDtypeStruct((e, 2 * _D), jnp.float32),
    )(z, w2, b2)


def _update(m, g, mask, h, wua, wub, ub1, uw2, ub2, lng, lnb):
    return pl.pallas_call(
        _update_body,
        grid=(_N // _NB,),
        in_specs=[_rows((_NB, _D)), _rows((_NB, _D)), _rows((_NB, 1)),
                  _rows((_NB, _D)), _full((_D, 2 * _D)), _full((_D, 2 * _D)),
                  _full((1, 2 * _D)), _full((2 * _D, _D)), _full((1, _D)),
                  _full((1, _D)), _full((1, _D))],
        out_specs=_rows((_NB, _D)),
        out_shape=jax.ShapeDtypeStruct((_N, _D), jnp.float32),
    )(m, g, mask, h, wua, wub, ub1, uw2, ub2, lng, lnb)


def kernel(node_embeddings, rel2_indices, rel1_indices, rel2_W1, rel2_b1,
           rel2_W2, rel2_b2, rel1_W1, rel1_b1, rel1_W2, rel1_b2, upd_W1,
           upd_b1, upd_W2, upd_b2, ln_g, ln_b):
    h = node_embeddings
    w1a, w1b = rel2_W1[:_D], rel2_W1[_D:]
    wua, wub = upd_W1[:_D], upd_W1[_D:]
    b1 = rel2_b1.reshape(1, -1)
    b2 = rel2_b2.reshape(1, -1)
    r1b1 = rel1_b1.reshape(1, -1)
    r1b2 = rel1_b2.reshape(1, -1)
    ub1 = upd_b1.reshape(1, -1)
    ub2 = upd_b2.reshape(1, -1)
    lng = ln_g.reshape(1, -1)
    lnb = ln_b.reshape(1, -1)
    src = rel2_indices[0::2]
    dst = rel2_indices[1::2]
    mask = jnp.zeros((_N, 1), jnp.float32).at[rel1_indices].set(1.0)
    perm = jnp.argsort(rel2_indices)
    sidx = rel2_indices[perm]
    for _ in range(_LAYERS):
        a, b, g = _prologue(h, w1a, w1b, b1, rel1_W1, r1b1, rel1_W2, r1b2)
        z = jnp.take(a, src, axis=0) + jnp.take(b, dst, axis=0)
        f = _edge_mlp(z, rel2_W2, b2)
        fp = jnp.take(f.reshape(-1, _D), perm, axis=0)
        m = jnp.full((_N, _D), -jnp.inf, jnp.float32).at[sidx].max(
            fp, indices_are_sorted=True)
        h = _update(m, g, mask, h, wua, wub, ub1, uw2=upd_W2, ub2=ub2,
                    lng=lng, lnb=lnb)
    return h


# speedup vs baseline: 1.1835x; 1.1835x over previous
import functools

import jax
import jax.numpy as jnp
from jax import lax
from jax.experimental import pallas as pl
from jax.experimental.pallas import tpu as pltpu
from jax.experimental.pallas import tpu_sc as plsc

_N = 10000
_D = 128
_LAYERS = 3
_NB = 1000
_EB = 2000

_NT = 32
_NPT = 313
_NPAD = _NT * _NPT
_ACC_ROWS = _NPT + 1
_CH = 128
_NSLOTS = 640000
_SPAD = _NSLOTS + 2 * _CH


def _prologue_body(h_ref, w1a_ref, w1b_ref, b1_ref, r1w1_ref, r1b1_ref,
                   r1w2_ref, r1b2_ref, a_ref, b_ref, g_ref):
    h = h_ref[...]
    a_ref[...] = jnp.dot(h, w1a_ref[...], preferred_element_type=jnp.float32) + b1_ref[...]
    b_ref[...] = jnp.dot(h, w1b_ref[...], preferred_element_type=jnp.float32)
    t = jax.nn.relu(jnp.dot(h, r1w1_ref[...], preferred_element_type=jnp.float32) + r1b1_ref[...])
    g_ref[...] = jnp.dot(t, r1w2_ref[...], preferred_element_type=jnp.float32) + r1b2_ref[...]


def _edge_body(z_ref, w2_ref, b2_ref, f_ref):
    f_ref[...] = (jnp.dot(jax.nn.relu(z_ref[...]), w2_ref[...],
                          preferred_element_type=jnp.float32) + b2_ref[...])


def _update_body(m_ref, g_ref, mask_ref, h_ref, wua_ref, wub_ref, ub1_ref,
                 uw2_ref, ub2_ref, lng_ref, lnb_ref, o_ref):
    m = m_ref[...]
    g = g_ref[...]
    mask = mask_ref[...]
    m = jnp.maximum(m, jnp.where(mask > 0.0, g, -jnp.inf))
    h = h_ref[...]
    x = h + m
    t = jax.nn.relu(jnp.dot(x, wua_ref[...], preferred_element_type=jnp.float32)
                    + jnp.dot(h, wub_ref[...], preferred_element_type=jnp.float32)
                    + ub1_ref[...])
    u = jnp.dot(t, uw2_ref[...], preferred_element_type=jnp.float32) + ub2_ref[...]
    mu = jnp.mean(u, axis=-1, keepdims=True)
    var = jnp.mean((u - mu) ** 2, axis=-1, keepdims=True)
    u = (u - mu) * jax.lax.rsqrt(var + 1e-5) * lng_ref[...] + lnb_ref[...]
    o_ref[...] = h + u


def _agg_body(f2_hbm, perm_hbm, sdst_hbm, bnd_hbm, out_hbm,
              idx_v, dst_v, lid_v, rows_v, acc_v, bnd_v, sem):
    c = lax.axis_index("c")
    s = lax.axis_index("s")
    wid = s * 2 + c
    base = wid * _NPT

    pltpu.sync_copy(bnd_hbm.at[wid], bnd_v)
    bv = bnd_v[...]
    start = bv[0]
    end = bv[8]
    start8 = (start // 8) * 8
    nch = (end - start8 + _CH - 1) // _CH

    neg = jnp.full((16,), -jnp.inf, jnp.float32)

    def init_row(r, _):
        for k in range(8):
            acc_v[pl.ds(r * _D + k * 16, 16)] = neg
        return 0

    lax.fori_loop(0, _ACC_ROWS, init_row, 0)

    def chunk(j, _):
        off = start8 + j * _CH
        pltpu.sync_copy(perm_hbm.at[pl.ds(off, _CH)], idx_v)
        pltpu.sync_copy(sdst_hbm.at[pl.ds(off, _CH)], dst_v)
        pltpu.async_copy(f2_hbm.at[idx_v], rows_v, sem).wait()
        for kb in range(_CH // 16):
            d = dst_v[pl.ds(kb * 16, 16)]
            l = d - base
            ok = (l >= 0) & (l < _NPT)
            lid_v[pl.ds(kb * 16, 16)] = jnp.where(ok, l, _NPT)

        def rmw(e, _):
            roff = lid_v[pl.ds(e, 16)][0] * _D
            for k in range(8):
                av = acc_v[pl.ds(roff + k * 16, 16)]
                rv = rows_v[e, pl.ds(k * 16, 16)]
                acc_v[pl.ds(roff + k * 16, 16)] = jnp.maximum(av, rv)
            return 0

        lax.fori_loop(0, _CH, rmw, 0)
        return 0

    lax.fori_loop(0, nch, chunk, 0)
    pltpu.sync_copy(acc_v.at[pl.ds(0, _NPT * _D)],
                    out_hbm.at[pl.ds(base * _D, _NPT * _D)])


def _sc_scatter_max(f2, perm, sdst, bnd):
    mesh = plsc.VectorSubcoreMesh(core_axis_name="c", subcore_axis_name="s")
    k = pl.kernel(
        _agg_body,
        mesh=mesh,
        out_type=jax.ShapeDtypeStruct((_NPAD * _D,), jnp.float32),
        scratch_types=[
            pltpu.VMEM((_CH,), jnp.int32),
            pltpu.VMEM((_CH,), jnp.int32),
            pltpu.VMEM((_CH + 16,), jnp.int32),
            pltpu.VMEM((_CH, _D), jnp.float32),
            pltpu.VMEM((_ACC_ROWS * _D,), jnp.float32),
            pltpu.VMEM((16,), jnp.int32),
            pltpu.SemaphoreType.DMA,
        ],
    )
    return k(f2, perm, sdst, bnd)


def _full(block):
    return pl.BlockSpec(block, lambda i: (0, 0))


def _rows(block):
    return pl.BlockSpec(block, lambda i: (i, 0))


def _prologue(h, w1a, w1b, b1, r1w1, r1b1, r1w2, r1b2):
    return pl.pallas_call(
        _prologue_body,
        grid=(_N // _NB,),
        in_specs=[_rows((_NB, _D)), _full((_D, 2 * _D)), _full((_D, 2 * _D)),
                  _full((1, 2 * _D)), _full((_D, _D)), _full((1, _D)),
                  _full((_D, _D)), _full((1, _D))],
        out_specs=[_rows((_NB, 2 * _D)), _rows((_NB, 2 * _D)), _rows((_NB, _D))],
        out_shape=[jax.ShapeDtypeStruct((_N, 2 * _D), jnp.float32),
                   jax.ShapeDtypeStruct((_N, 2 * _D), jnp.float32),
                   jax.ShapeDtypeStruct((_N, _D), jnp.float32)],
    )(h, w1a, w1b, b1, r1w1, r1b1, r1w2, r1b2)


def _edge_mlp(z, w2, b2):
    e = z.shape[0]
    return pl.pallas_call(
        _edge_body,
        grid=(e // _EB,),
        in_specs=[_rows((_EB, 2 * _D)), _full((2 * _D, 2 * _D)), _full((1, 2 * _D))],
        out_specs=_rows((_EB, 2 * _D)),
        out_shape=jax.ShapeDtypeStruct((e, 2 * _D), jnp.float32),
    )(z, w2, b2)


def _update(m, g, mask, h, wua, wub, ub1, uw2, ub2, lng, lnb):
    return pl.pallas_call(
        _update_body,
        grid=(_N // _NB,),
        in_specs=[_rows((_NB, _D)), _rows((_NB, _D)), _rows((_NB, 1)),
                  _rows((_NB, _D)), _full((_D, 2 * _D)), _full((_D, 2 * _D)),
                  _full((1, 2 * _D)), _full((2 * _D, _D)), _full((1, _D)),
                  _full((1, _D)), _full((1, _D))],
        out_specs=_rows((_NB, _D)),
        out_shape=jax.ShapeDtypeStruct((_N, _D), jnp.float32),
    )(m, g, mask, h, wua, wub, ub1, uw2, ub2, lng, lnb)


def kernel(node_embeddings, rel2_indices, rel1_indices, rel2_W1, rel2_b1,
           rel2_W2, rel2_b2, rel1_W1, rel1_b1, rel1_W2, rel1_b2, upd_W1,
           upd_b1, upd_W2, upd_b2, ln_g, ln_b):
    h = node_embeddings
    w1a, w1b = rel2_W1[:_D], rel2_W1[_D:]
    wua, wub = upd_W1[:_D], upd_W1[_D:]
    b1 = rel2_b1.reshape(1, -1)
    b2 = rel2_b2.reshape(1, -1)
    r1b1 = rel1_b1.reshape(1, -1)
    r1b2 = rel1_b2.reshape(1, -1)
    ub1 = upd_b1.reshape(1, -1)
    ub2 = upd_b2.reshape(1, -1)
    lng = ln_g.reshape(1, -1)
    lnb = ln_b.reshape(1, -1)
    src = rel2_indices[0::2]
    dst = rel2_indices[1::2]
    mask = jnp.zeros((_N, 1), jnp.float32).at[rel1_indices].set(1.0)
    perm = jnp.argsort(rel2_indices).astype(jnp.int32)
    sdst = jnp.take(rel2_indices, perm)
    perm = jnp.concatenate([perm, jnp.zeros((_SPAD - _NSLOTS,), jnp.int32)])
    sdst = jnp.concatenate(
        [sdst, jnp.full((_SPAD - _NSLOTS,), jnp.int32(1 << 20))])
    tile_starts = jnp.arange(_NT + 1, dtype=jnp.int32) * _NPT
    bnds = jnp.searchsorted(sdst[:_NSLOTS], tile_starts).astype(jnp.int32)
    bnd = jnp.concatenate([
        jnp.repeat(bnds[:_NT, None], 8, axis=1),
        jnp.repeat(bnds[1:, None], 8, axis=1)], axis=1)
    for _ in range(_LAYERS):
        a, b, g = _prologue(h, w1a, w1b, b1, rel1_W1, r1b1, rel1_W2, r1b2)
        z = jnp.take(a, src, axis=0) + jnp.take(b, dst, axis=0)
        f = _edge_mlp(z, rel2_W2, b2)
        mp = _sc_scatter_max(f.reshape(-1, _D), perm, sdst, bnd)
        m = mp.reshape(_NPAD, _D)[:_N]
        h = _update(m, g, mask, h, wua, wub, ub1, uw2=upd_W2, ub2=ub2,
                    lng=lng, lnb=lnb)
    return h


# speedup vs baseline: 1.6368x; 1.3830x over previous
import functools

import jax
import jax.numpy as jnp
from jax import lax
from jax.experimental import pallas as pl
from jax.experimental.pallas import tpu as pltpu
from jax.experimental.pallas import tpu_sc as plsc

_N = 10000
_D = 128
_LAYERS = 3
_NB = 1000
_EB = 2000

_NT = 32
_NPT = 313
_NPAD = _NT * _NPT
_ACC_ROWS = _NPT + 1
_CH = 128
_NSLOTS = 640000
_SPAD = _NSLOTS + 2 * _CH


def _prologue_body(h_ref, w1a_ref, w1b_ref, b1_ref, r1w1_ref, r1b1_ref,
                   r1w2_ref, r1b2_ref, a_ref, b_ref, g_ref):
    h = h_ref[...]
    a_ref[...] = jnp.dot(h, w1a_ref[...], preferred_element_type=jnp.float32) + b1_ref[...]
    b_ref[...] = jnp.dot(h, w1b_ref[...], preferred_element_type=jnp.float32)
    t = jax.nn.relu(jnp.dot(h, r1w1_ref[...], preferred_element_type=jnp.float32) + r1b1_ref[...])
    g_ref[...] = jnp.dot(t, r1w2_ref[...], preferred_element_type=jnp.float32) + r1b2_ref[...]


def _edge_body(z_ref, w2_ref, b2_ref, f_ref):
    f_ref[...] = (jnp.dot(jax.nn.relu(z_ref[...]), w2_ref[...],
                          preferred_element_type=jnp.float32) + b2_ref[...])


def _update_body(m_ref, g_ref, mask_ref, h_ref, wua_ref, wub_ref, ub1_ref,
                 uw2_ref, ub2_ref, lng_ref, lnb_ref, o_ref):
    m = m_ref[...]
    g = g_ref[...]
    mask = mask_ref[...]
    m = jnp.maximum(m, jnp.where(mask > 0.0, g, -jnp.inf))
    h = h_ref[...]
    x = h + m
    t = jax.nn.relu(jnp.dot(x, wua_ref[...], preferred_element_type=jnp.float32)
                    + jnp.dot(h, wub_ref[...], preferred_element_type=jnp.float32)
                    + ub1_ref[...])
    u = jnp.dot(t, uw2_ref[...], preferred_element_type=jnp.float32) + ub2_ref[...]
    mu = jnp.mean(u, axis=-1, keepdims=True)
    var = jnp.mean((u - mu) ** 2, axis=-1, keepdims=True)
    u = (u - mu) * jax.lax.rsqrt(var + 1e-5) * lng_ref[...] + lnb_ref[...]
    o_ref[...] = h + u


def _agg_body(f2_hbm, perm_hbm, sdst_hbm, bnd_hbm, out_hbm,
              idx_v, dst_v, lid_v, rows_v, acc_v, bnd_v, sem):
    c = lax.axis_index("c")
    s = lax.axis_index("s")
    wid = s * 2 + c
    base = wid * _NPT

    pltpu.sync_copy(bnd_hbm.at[wid], bnd_v)
    bv = bnd_v[...]
    start = bv[0]
    end = bv[8]
    start8 = (start // 8) * 8
    nch = (end - start8 + _CH - 1) // _CH

    neg = jnp.full((16,), -jnp.inf, jnp.float32)

    def init_row(r, _):
        for k in range(8):
            acc_v[pl.ds(r * _D + k * 16, 16)] = neg
        return 0

    lax.fori_loop(0, _ACC_ROWS, init_row, 0)

    def chunk(j, _):
        off = start8 + j * _CH
        pltpu.sync_copy(perm_hbm.at[pl.ds(off, _CH)], idx_v)
        pltpu.sync_copy(sdst_hbm.at[pl.ds(off, _CH)], dst_v)
        pltpu.async_copy(f2_hbm.at[idx_v], rows_v, sem).wait()
        for kb in range(_CH // 16):
            d = dst_v[pl.ds(kb * 16, 16)]
            l = d - base
            ok = (l >= 0) & (l < _NPT)
            lid_v[pl.ds(kb * 16, 16)] = jnp.where(ok, l, _NPT)

        def rmw(e, _):
            roff = lid_v[pl.ds(e, 16)][0] * _D
            for k in range(8):
                av = acc_v[pl.ds(roff + k * 16, 16)]
                rv = rows_v[e, pl.ds(k * 16, 16)]
                acc_v[pl.ds(roff + k * 16, 16)] = jnp.maximum(av, rv)
            return 0

        lax.fori_loop(0, _CH, rmw, 0)
        return 0

    lax.fori_loop(0, nch, chunk, 0)
    pltpu.sync_copy(acc_v.at[pl.ds(0, _NPT * _D)],
                    out_hbm.at[pl.ds(base * _D, _NPT * _D)])


def _sc_scatter_max(f2, perm, sdst, bnd):
    mesh = plsc.VectorSubcoreMesh(core_axis_name="c", subcore_axis_name="s")
    k = pl.kernel(
        _agg_body,
        mesh=mesh,
        out_type=jax.ShapeDtypeStruct((_NPAD * _D,), jnp.float32),
        scratch_types=[
            pltpu.VMEM((_CH,), jnp.int32),
            pltpu.VMEM((_CH,), jnp.int32),
            pltpu.VMEM((_CH + 16,), jnp.int32),
            pltpu.VMEM((_CH, _D), jnp.float32),
            pltpu.VMEM((_ACC_ROWS * _D,), jnp.float32),
            pltpu.VMEM((16,), jnp.int32),
            pltpu.SemaphoreType.DMA,
        ],
    )
    return k(f2, perm, sdst, bnd)


_E2 = 320000
_EPT = _E2 // _NT
_GC = 80


def _gather_body(a_hbm, b_hbm, src_hbm, dst_hbm, z_hbm,
                 sidx_v, didx_v, arows_v, brows_v, sem_a, sem_b):
    c = lax.axis_index("c")
    s = lax.axis_index("s")
    wid = s * 2 + c
    ebase = wid * _EPT

    def chunk(j, _):
        off = ebase + j * _GC
        pltpu.sync_copy(src_hbm.at[pl.ds(off, _GC)], sidx_v)
        pltpu.sync_copy(dst_hbm.at[pl.ds(off, _GC)], didx_v)
        ca = pltpu.async_copy(a_hbm.at[sidx_v], arows_v, sem_a)
        cb = pltpu.async_copy(b_hbm.at[didx_v], brows_v, sem_b)
        ca.wait()
        cb.wait()

        def addrow(r, _):
            for k in range(16):
                av = arows_v[r, pl.ds(k * 16, 16)]
                bv = brows_v[r, pl.ds(k * 16, 16)]
                arows_v[r, pl.ds(k * 16, 16)] = av + bv
            return 0

        lax.fori_loop(0, _GC, addrow, 0)
        pltpu.sync_copy(arows_v, z_hbm.at[pl.ds(off, _GC)])
        return 0

    lax.fori_loop(0, _EPT // _GC, chunk, 0)


def _sc_gather_add(a, b, src, dst):
    mesh = plsc.VectorSubcoreMesh(core_axis_name="c", subcore_axis_name="s")
    k = pl.kernel(
        _gather_body,
        mesh=mesh,
        out_type=jax.ShapeDtypeStruct((_E2, 2 * _D), jnp.float32),
        scratch_types=[
            pltpu.VMEM((_GC,), jnp.int32),
            pltpu.VMEM((_GC,), jnp.int32),
            pltpu.VMEM((_GC, 2 * _D), jnp.float32),
            pltpu.VMEM((_GC, 2 * _D), jnp.float32),
            pltpu.SemaphoreType.DMA,
            pltpu.SemaphoreType.DMA,
        ],
    )
    return k(a, b, src, dst)


def _full(block):
    return pl.BlockSpec(block, lambda i: (0, 0))


def _rows(block):
    return pl.BlockSpec(block, lambda i: (i, 0))


def _prologue(h, w1a, w1b, b1, r1w1, r1b1, r1w2, r1b2):
    return pl.pallas_call(
        _prologue_body,
        grid=(_N // _NB,),
        in_specs=[_rows((_NB, _D)), _full((_D, 2 * _D)), _full((_D, 2 * _D)),
                  _full((1, 2 * _D)), _full((_D, _D)), _full((1, _D)),
                  _full((_D, _D)), _full((1, _D))],
        out_specs=[_rows((_NB, 2 * _D)), _rows((_NB, 2 * _D)), _rows((_NB, _D))],
        out_shape=[jax.ShapeDtypeStruct((_N, 2 * _D), jnp.float32),
                   jax.ShapeDtypeStruct((_N, 2 * _D), jnp.float32),
                   jax.ShapeDtypeStruct((_N, _D), jnp.float32)],
    )(h, w1a, w1b, b1, r1w1, r1b1, r1w2, r1b2)


def _edge_mlp(z, w2, b2):
    e = z.shape[0]
    return pl.pallas_call(
        _edge_body,
        grid=(e // _EB,),
        in_specs=[_rows((_EB, 2 * _D)), _full((2 * _D, 2 * _D)), _full((1, 2 * _D))],
        out_specs=_rows((_EB, 2 * _D)),
        out_shape=jax.ShapeDtypeStruct((e, 2 * _D), jnp.float32),
    )(z, w2, b2)


def _update(m, g, mask, h, wua, wub, ub1, uw2, ub2, lng, lnb):
    return pl.pallas_call(
        _update_body,
        grid=(_N // _NB,),
        in_specs=[_rows((_NB, _D)), _rows((_NB, _D)), _rows((_NB, 1)),
                  _rows((_NB, _D)), _full((_D, 2 * _D)), _full((_D, 2 * _D)),
                  _full((1, 2 * _D)), _full((2 * _D, _D)), _full((1, _D)),
                  _full((1, _D)), _full((1, _D))],
        out_specs=_rows((_NB, _D)),
        out_shape=jax.ShapeDtypeStruct((_N, _D), jnp.float32),
    )(m, g, mask, h, wua, wub, ub1, uw2, ub2, lng, lnb)


def kernel(node_embeddings, rel2_indices, rel1_indices, rel2_W1, rel2_b1,
           rel2_W2, rel2_b2, rel1_W1, rel1_b1, rel1_W2, rel1_b2, upd_W1,
           upd_b1, upd_W2, upd_b2, ln_g, ln_b):
    h = node_embeddings
    w1a, w1b = rel2_W1[:_D], rel2_W1[_D:]
    wua, wub = upd_W1[:_D], upd_W1[_D:]
    b1 = rel2_b1.reshape(1, -1)
    b2 = rel2_b2.reshape(1, -1)
    r1b1 = rel1_b1.reshape(1, -1)
    r1b2 = rel1_b2.reshape(1, -1)
    ub1 = upd_b1.reshape(1, -1)
    ub2 = upd_b2.reshape(1, -1)
    lng = ln_g.reshape(1, -1)
    lnb = ln_b.reshape(1, -1)
    src = rel2_indices[0::2]
    dst = rel2_indices[1::2]
    mask = jnp.zeros((_N, 1), jnp.float32).at[rel1_indices].set(1.0)
    perm = jnp.argsort(rel2_indices).astype(jnp.int32)
    sdst = jnp.take(rel2_indices, perm)
    perm = jnp.concatenate([perm, jnp.zeros((_SPAD - _NSLOTS,), jnp.int32)])
    sdst = jnp.concatenate(
        [sdst, jnp.full((_SPAD - _NSLOTS,), jnp.int32(1 << 20))])
    tile_starts = jnp.arange(_NT + 1, dtype=jnp.int32) * _NPT
    bnds = jnp.searchsorted(sdst[:_NSLOTS], tile_starts).astype(jnp.int32)
    bnd = jnp.concatenate([
        jnp.repeat(bnds[:_NT, None], 8, axis=1),
        jnp.repeat(bnds[1:, None], 8, axis=1)], axis=1)
    for _ in range(_LAYERS):
        a, b, g = _prologue(h, w1a, w1b, b1, rel1_W1, r1b1, rel1_W2, r1b2)
        z = _sc_gather_add(a, b, src, dst)
        f = _edge_mlp(z, rel2_W2, b2)
        mp = _sc_scatter_max(f.reshape(-1, _D), perm, sdst, bnd)
        m = mp.reshape(_NPAD, _D)[:_N]
        h = _update(m, g, mask, h, wua, wub, ub1, uw2=upd_W2, ub2=ub2,
                    lng=lng, lnb=lnb)
    return h
